# Initial kernel scaffold; baseline (speedup 1.0000x reference)
import jax
import jax.numpy as jnp
from jax.experimental import pallas as pl


def kernel(x, mask):
    raise NotImplementedError("write your pallas kernel here")


# speedup vs baseline: 6.2821x; 6.2821x over previous
import functools

import jax
import jax.numpy as jnp
from jax import lax
from jax.experimental import pallas as pl
from jax.experimental.pallas import tpu as pltpu
from jax.experimental.pallas import tpu_sc as plsc

B, C, F = 1024, 50, 1000
NTRIM = 5
KEEP = C - 2 * NTRIM
L = 16
FP = 1008
NC, NS = 2, 16
NW = NC * NS
BPW = B // NW
NFT = FP // L


def _tm_body(x_hbm, out_hbm, xbuf, obuf):
    cid = lax.axis_index("c")
    sid = lax.axis_index("s")
    wid = sid * NC + cid
    base = wid * BPW

    @pl.loop(0, BPW)
    def _batch(i):
        b = base + i
        pltpu.sync_copy(x_hbm.at[b], xbuf.at[:, pl.ds(0, F)])

        @pl.loop(0, NFT)
        def _ftile(ft):
            off = pl.multiple_of(ft * L, L)
            neg = jnp.full((L,), -jnp.inf, jnp.float32)
            pos = jnp.full((L,), jnp.inf, jnp.float32)
            top = [neg] * NTRIM
            bot = [pos] * NTRIM
            tot = jnp.zeros((L,), jnp.float32)
            for c in range(C):
                v = xbuf[c, pl.ds(off, L)]
                tot = tot + v
                u = v
                for k in range(NTRIM):
                    hi = jnp.maximum(top[k], u)
                    u = jnp.minimum(top[k], u)
                    top[k] = hi
                w = v
                for k in range(NTRIM):
                    lo = jnp.minimum(bot[k], w)
                    w = jnp.maximum(bot[k], w)
                    bot[k] = lo
            for k in range(NTRIM):
                tot = tot - top[k] - bot[k]
            obuf[pl.ds(off, L)] = tot * (1.0 / KEEP)

        pltpu.sync_copy(obuf.at[pl.ds(0, F)], out_hbm.at[b])


def kernel(x, mask):
    del mask
    mesh = plsc.VectorSubcoreMesh(core_axis_name="c", subcore_axis_name="s")
    tm = pl.kernel(
        _tm_body,
        out_type=jax.ShapeDtypeStruct((B, F), jnp.float32),
        mesh=mesh,
        scratch_types=[
            pltpu.VMEM((C, FP), jnp.float32),
            pltpu.VMEM((FP,), jnp.float32),
        ],
        compiler_params=pltpu.CompilerParams(use_tc_tiling_on_sc=False),
    )
    return tm(x)
